# baseline (speedup 1.0000x reference)
import functools

import jax
import jax.numpy as jnp
from jax import lax
from jax.experimental import pallas as pl
from jax.experimental.pallas import tpu as pltpu
from jax.experimental.pallas import tpu_sc as plsc

B, T = 1024, 50
V, E, H = 100000, 128, 256
G = 3 * H


@functools.lru_cache(maxsize=1)
def _make_sc_gather():
    NC, NS = 2, 16
    NW = NC * NS
    TB = T * B
    per_w = TB // NW
    CH = 80
    n_ch = per_w // CH

    mesh = plsc.VectorSubcoreMesh(core_axis_name="c", subcore_axis_name="s")

    @functools.partial(
        pl.kernel,
        mesh=mesh,
        out_type=jax.ShapeDtypeStruct((TB, E), jnp.float32),
        scratch_types=[
            pltpu.VMEM((per_w,), jnp.int32),
            pltpu.VMEM((CH, E), jnp.float32),
            pltpu.VMEM((CH, E), jnp.float32),
            pltpu.SemaphoreType.DMA,
            pltpu.SemaphoreType.DMA,
        ],
    )
    def gather_k(table_hbm, idx_hbm, out_hbm, idx_v, rows0, rows1, s0, s1):
        wid = lax.axis_index("s") * NC + lax.axis_index("c")
        base = wid * per_w
        pltpu.sync_copy(idx_hbm.at[pl.ds(base, per_w)], idx_v)

        def fire(i, buf, sem):
            pltpu.async_copy(table_hbm.at[idx_v.at[pl.ds(i * CH, CH)]], buf, sem)

        def drain(buf, sem):
            pltpu.make_async_copy(table_hbm.at[idx_v.at[pl.ds(0, CH)]], buf, sem).wait()

        fire(0, rows0, s0)
        fire(1, rows1, s1)

        def body(j, carry):
            i0 = j * 2
            drain(rows0, s0)
            pltpu.sync_copy(rows0, out_hbm.at[pl.ds(base + i0 * CH, CH)])

            @pl.when(i0 + 2 < n_ch)
            def _():
                fire(i0 + 2, rows0, s0)

            drain(rows1, s1)
            pltpu.sync_copy(rows1, out_hbm.at[pl.ds(base + (i0 + 1) * CH, CH)])

            @pl.when(i0 + 3 < n_ch)
            def _():
                fire(i0 + 3, rows1, s1)

            return carry

        lax.fori_loop(0, n_ch // 2, body, 0)

    return gather_k


def _gru_body(e_ref, wih0, whh0, bih0, bhh0, wih1, whh1, bih1, bhh1,
              y_ref, hid_ref, h0_s, h1_s):
    t = pl.program_id(0)

    @pl.when(t == 0)
    def _():
        h0_s[...] = jnp.zeros_like(h0_s)
        h1_s[...] = jnp.zeros_like(h1_s)

    def cell(xt_b, h, wihT, whhT, bih, bhh):
        gi = lax.dot_general(xt_b, wihT, (((1,), (0,)), ((), ())),
                             preferred_element_type=jnp.float32) + bih
        gh = lax.dot_general(h.astype(jnp.bfloat16), whhT, (((1,), (0,)), ((), ())),
                             preferred_element_type=jnp.float32) + bhh
        r = jax.nn.sigmoid(gi[:, :H] + gh[:, :H])
        z = jax.nn.sigmoid(gi[:, H:2 * H] + gh[:, H:2 * H])
        n = jnp.tanh(gi[:, 2 * H:] + r * gh[:, 2 * H:])
        return (1.0 - z) * n + z * h

    h0 = cell(e_ref[0].astype(jnp.bfloat16), h0_s[...],
              wih0[...], whh0[...], bih0[...], bhh0[...])
    h0_s[...] = h0
    h1 = cell(h0.astype(jnp.bfloat16), h1_s[...],
              wih1[...], whh1[...], bih1[...], bhh1[...])
    h1_s[...] = h1
    y_ref[0] = h1

    @pl.when(t == T - 1)
    def _():
        hid_ref[0] = h0
        hid_ref[1] = h1


def _gru2(e_tbE, wih0T, whh0T, bih0, bhh0, wih1T, whh1T, bih1, bhh1):
    full = lambda shape: pl.BlockSpec(shape, lambda t: tuple(0 for _ in shape))
    y, hid = pl.pallas_call(
        _gru_body,
        grid=(T,),
        in_specs=[
            pl.BlockSpec((1, B, E), lambda t: (t, 0, 0)),
            full((E, G)), full((H, G)), full((1, G)), full((1, G)),
            full((H, G)), full((H, G)), full((1, G)), full((1, G)),
        ],
        out_specs=[
            pl.BlockSpec((1, B, H), lambda t: (t, 0, 0)),
            pl.BlockSpec((2, B, H), lambda t: (0, 0, 0)),
        ],
        out_shape=[
            jax.ShapeDtypeStruct((T, B, H), jnp.float32),
            jax.ShapeDtypeStruct((2, B, H), jnp.float32),
        ],
        scratch_shapes=[
            pltpu.VMEM((B, H), jnp.float32),
            pltpu.VMEM((B, H), jnp.float32),
        ],
    )(e_tbE, wih0T, whh0T, bih0, bhh0, wih1T, whh1T, bih1, bhh1)
    return y, hid


def kernel(x, emb, W_ih_l0, W_hh_l0, b_ih_l0, b_hh_l0,
           W_ih_l1, W_hh_l1, b_ih_l1, b_hh_l1):
    idx = x.T.reshape(-1).astype(jnp.int32)
    e = _make_sc_gather()(emb, idx)
    e = e.reshape(T, B, E)

    bf = jnp.bfloat16
    y, hid = _gru2(
        e,
        W_ih_l0.T.astype(bf), W_hh_l0.T.astype(bf),
        b_ih_l0.reshape(1, G), b_hh_l0.reshape(1, G),
        W_ih_l1.T.astype(bf), W_hh_l1.T.astype(bf),
        b_ih_l1.reshape(1, G), b_hh_l1.reshape(1, G),
    )
    return jnp.swapaxes(y, 0, 1), hid


# speedup vs baseline: 1.1502x; 1.1502x over previous
import functools

import jax
import jax.numpy as jnp
from jax import lax
from jax.experimental import pallas as pl
from jax.experimental.pallas import tpu as pltpu
from jax.experimental.pallas import tpu_sc as plsc

B, T = 1024, 50
V, E, H = 100000, 128, 256
G = 3 * H


@functools.lru_cache(maxsize=1)
def _make_sc_gather():
    NC, NS = 2, 16
    NW = NC * NS
    TB = T * B
    per_w = TB // NW
    CH = 80
    n_ch = per_w // CH

    mesh = plsc.VectorSubcoreMesh(core_axis_name="c", subcore_axis_name="s")

    @functools.partial(
        pl.kernel,
        mesh=mesh,
        out_type=jax.ShapeDtypeStruct((TB, E), jnp.float32),
        scratch_types=[
            pltpu.VMEM((per_w,), jnp.int32),
            pltpu.VMEM((CH, E), jnp.float32),
            pltpu.VMEM((CH, E), jnp.float32),
            pltpu.SemaphoreType.DMA,
            pltpu.SemaphoreType.DMA,
        ],
    )
    def gather_k(table_hbm, idx_hbm, out_hbm, idx_v, rows0, rows1, s0, s1):
        wid = lax.axis_index("s") * NC + lax.axis_index("c")
        base = wid * per_w
        pltpu.sync_copy(idx_hbm.at[pl.ds(base, per_w)], idx_v)

        def fire(i, buf, sem):
            pltpu.async_copy(table_hbm.at[idx_v.at[pl.ds(i * CH, CH)]], buf, sem)

        def drain(buf, sem):
            pltpu.make_async_copy(table_hbm.at[idx_v.at[pl.ds(0, CH)]], buf, sem).wait()

        fire(0, rows0, s0)
        fire(1, rows1, s1)

        def body(j, carry):
            i0 = j * 2
            drain(rows0, s0)
            pltpu.sync_copy(rows0, out_hbm.at[pl.ds(base + i0 * CH, CH)])

            @pl.when(i0 + 2 < n_ch)
            def _():
                fire(i0 + 2, rows0, s0)

            drain(rows1, s1)
            pltpu.sync_copy(rows1, out_hbm.at[pl.ds(base + (i0 + 1) * CH, CH)])

            @pl.when(i0 + 3 < n_ch)
            def _():
                fire(i0 + 3, rows1, s1)

            return carry

        lax.fori_loop(0, n_ch // 2, body, 0)

    return gather_k


NCHUNK = 4
BC = B // NCHUNK
TS = 10


def _gru_body(e_ref, wrz0, win0, whn0, brz0, bin0, bhn0,
              wrz1, win1, whn1, brz1, bin1, bhn1,
              y_ref, hid_ref, *scr):
    t = pl.program_id(0)
    bf = jnp.bfloat16
    h0fs = scr[0::4]
    h1fs = scr[1::4]
    a0s = scr[2::4]
    a1s = scr[3::4]

    @pl.when(t == 0)
    def _():
        for c in range(NCHUNK):
            h0fs[c][...] = jnp.zeros_like(h0fs[c])
            h1fs[c][...] = jnp.zeros_like(h1fs[c])
            a0s[c][:, E:] = jnp.zeros((BC, H), bf)
            a1s[c][:, H:] = jnp.zeros((BC, H), bf)

    def dot(a, b):
        return lax.dot_general(a, b, (((1,), (0,)), ((), ())),
                               preferred_element_type=jnp.float32)

    def dots(a_ref, K, wrz, win, whn, brz, bin_, bhn):
        s = dot(a_ref[...], wrz[...]) + brz[...]
        gin = dot(a_ref[:, :K], win[...]) + bin_[...]
        ghn = dot(a_ref[:, K:], whn[...]) + bhn[...]
        return s, gin, ghn

    def gates(s, gin, ghn, hf_ref):
        r = 0.5 * jnp.tanh(s[:, :H]) + 0.5
        z = 0.5 * jnp.tanh(s[:, H:]) + 0.5
        n = jnp.tanh(gin + r * ghn)
        hnew = z * (hf_ref[...] - n) + n
        hf_ref[...] = hnew
        return hnew

    W0 = (wrz0, win0, whn0, brz0, bin0, bhn0)
    W1 = (wrz1, win1, whn1, brz1, bin1, bhn1)

    h0 = [None] * NCHUNK
    h1 = [None] * NCHUNK
    for u in range(TS):
        for c in range(NCHUNK):
            rows = pl.ds(c * BC, BC)
            a0s[c][:, :E] = e_ref[u, rows, :].astype(bf)

        d0 = [dots(a0s[c], E, *W0) for c in range(NCHUNK)]
        d1 = [None] * NCHUNK
        for c in range(NCHUNK):
            h0[c] = gates(*d0[c], h0fs[c])
            h0b = h0[c].astype(bf)
            a0s[c][:, E:] = h0b
            a1s[c][:, :H] = h0b
            d1[c] = dots(a1s[c], H, *W1)
        for c in range(NCHUNK):
            h1[c] = gates(*d1[c], h1fs[c])
            a1s[c][:, H:] = h1[c].astype(bf)
            y_ref[u, pl.ds(c * BC, BC), :] = h1[c]

    @pl.when(t == T // TS - 1)
    def _():
        for c in range(NCHUNK):
            rows = pl.ds(c * BC, BC)
            hid_ref[0, rows, :] = h0[c]
            hid_ref[1, rows, :] = h1[c]


def _gru2(e_tbE, *weights):
    full = lambda shape: pl.BlockSpec(shape, lambda t: tuple(0 for _ in shape))
    w_specs = [full(w.shape) for w in weights]
    y, hid = pl.pallas_call(
        _gru_body,
        grid=(T // TS,),
        in_specs=[pl.BlockSpec((TS, B, E), lambda t: (t, 0, 0))] + w_specs,
        out_specs=[
            pl.BlockSpec((TS, B, H), lambda t: (t, 0, 0)),
            pl.BlockSpec((2, B, H), lambda t: (0, 0, 0)),
        ],
        out_shape=[
            jax.ShapeDtypeStruct((T, B, H), jnp.float32),
            jax.ShapeDtypeStruct((2, B, H), jnp.float32),
        ],
        scratch_shapes=[
            s for _ in range(NCHUNK) for s in (
                pltpu.VMEM((BC, H), jnp.float32),
                pltpu.VMEM((BC, H), jnp.float32),
                pltpu.VMEM((BC, E + H), jnp.bfloat16),
                pltpu.VMEM((BC, 2 * H), jnp.bfloat16),
            )
        ],
    )(e_tbE, *weights)
    return y, hid


def kernel(x, emb, W_ih_l0, W_hh_l0, b_ih_l0, b_hh_l0,
           W_ih_l1, W_hh_l1, b_ih_l1, b_hh_l1):
    idx = x.T.reshape(-1).astype(jnp.int32)
    e = _make_sc_gather()(emb, idx)
    e = e.reshape(T, B, E)

    bf = jnp.bfloat16

    def layer_weights(Wih, Whh, bih, bhh):
        wihT, whhT = Wih.T, Whh.T
        wrz = (0.5 * jnp.concatenate([wihT[:, :2 * H], whhT[:, :2 * H]],
                                     axis=0)).astype(bf)
        win = wihT[:, 2 * H:].astype(bf)
        whn = whhT[:, 2 * H:].astype(bf)
        brz = (0.5 * (bih + bhh))[:2 * H].reshape(1, 2 * H)
        bin_ = bih[2 * H:].reshape(1, H)
        bhn = bhh[2 * H:].reshape(1, H)
        return wrz, win, whn, brz, bin_, bhn

    y, hid = _gru2(
        e,
        *layer_weights(W_ih_l0, W_hh_l0, b_ih_l0, b_hh_l0),
        *layer_weights(W_ih_l1, W_hh_l1, b_ih_l1, b_hh_l1),
    )
    return jnp.swapaxes(y, 0, 1), hid
